# baseline (speedup 1.0000x reference)
import functools

import jax
import jax.numpy as jnp
from jax import lax
from jax.experimental import pallas as pl
from jax.experimental.pallas import tpu as pltpu
from jax.experimental.pallas import tpu_sc as plsc

NUM_CORES = 2
NUM_SUBCORES = 16
LANES = 16
NW = NUM_CORES * NUM_SUBCORES
BATCH = 16384
EDIM = 32
NROW = 1000000
WIDE = 128
PACK = WIDE // EDIM
WROWS = NROW * EDIM // WIDE
CW = 512
NFULL = NROW // CW
TAILW = 128
TAIL0 = NROW - TAILW
B_PER_W = BATCH // NW
CHUNK = 128
NCHUNK = B_PER_W // CHUNK
GROUPS = CHUNK // LANES


def _sc_relayout(ut_t, it_t, tail_u, tail_i):
    mesh = plsc.VectorSubcoreMesh(core_axis_name="c", subcore_axis_name="s")
    out_sds = jax.ShapeDtypeStruct((WROWS, WIDE), jnp.float32)

    @functools.partial(
        pl.kernel,
        out_type=(out_sds, out_sds),
        mesh=mesh,
        compiler_params=pltpu.CompilerParams(
            needs_layout_passes=False, use_tc_tiling_on_sc=True),
        scratch_types=[
            pltpu.VMEM((EDIM, CW), jnp.float32),
            pltpu.VMEM((CW // PACK, WIDE), jnp.float32),
            pltpu.VMEM((EDIM, TAILW), jnp.float32),
            pltpu.SemaphoreType.DMA,
        ],
    )
    def run(ut_hbm, it_hbm, tu_hbm, ti_hbm, uw_hbm, iw_hbm,
            in_v, out_v, in_t, sem):
        wid = lax.axis_index("s") * NUM_CORES + lax.axis_index("c")
        lane = lax.iota(jnp.int32, LANES)
        qbase = lane // PACK
        cmod = (lane % PACK) * EDIM

        for src_hbm, dst_hbm, t_hbm in (
                (ut_hbm, uw_hbm, tu_hbm), (it_hbm, iw_hbm, ti_hbm)):

            def chunk_body(k, carry, src_hbm=src_hbm, dst_hbm=dst_hbm):
                c = k * NW + wid

                @pl.when(c < NFULL)
                def _():
                    r0 = pl.multiple_of(c * CW, CW)
                    pltpu.async_copy(
                        src_hbm.at[:, pl.ds(r0, CW)], in_v, sem).wait()

                    def tbody(g, carry2):
                        rowv = 4 * g + qbase
                        vals0 = in_v[0, pl.ds(pl.multiple_of(g * LANES, LANES), LANES)]
                        s = pl.ds(pl.multiple_of(g * LANES, LANES), LANES)
                        for di in range(EDIM):
                            vals = in_v[di, s] if di else vals0
                            plsc.store_scatter(out_v, [rowv, cmod + di], vals)
                        return carry2

                    lax.fori_loop(0, CW // LANES, tbody, 0)
                    pltpu.async_copy(
                        out_v, dst_hbm.at[pl.ds(c * (CW // PACK), CW // PACK)],
                        sem).wait()

                return carry

            lax.fori_loop(0, NFULL // NW + 1, chunk_body, 0)

            @pl.when(wid == 0)
            def _(dst_hbm=dst_hbm, t_hbm=t_hbm):
                pltpu.async_copy(t_hbm, in_t, sem).wait()

                def tbody(g, carry2):
                    rowv = 4 * g + qbase
                    s = pl.ds(pl.multiple_of(g * LANES, LANES), LANES)
                    for di in range(EDIM):
                        plsc.store_scatter(out_v, [rowv, cmod + di], in_t[di, s])
                    return carry2

                lax.fori_loop(0, TAILW // LANES, tbody, 0)
                pltpu.async_copy(
                    out_v.at[pl.ds(0, TAILW // PACK)],
                    dst_hbm.at[pl.ds(TAIL0 // PACK, TAILW // PACK)], sem).wait()

    return run(ut_t, it_t, tail_u, tail_i)


def _sc_pred_diff(u, i, j, tw_u, tw_i):
    mesh = plsc.VectorSubcoreMesh(core_axis_name="c", subcore_axis_name="s")

    @functools.partial(
        pl.kernel,
        out_type=jax.ShapeDtypeStruct((BATCH // CHUNK, CHUNK), jnp.float32),
        mesh=mesh,
        compiler_params=pltpu.CompilerParams(
            needs_layout_passes=False, use_tc_tiling_on_sc=False),
        scratch_types=[
            pltpu.VMEM((NCHUNK, CHUNK), jnp.int32),
            pltpu.VMEM((NCHUNK, CHUNK), jnp.int32),
            pltpu.VMEM((NCHUNK, CHUNK), jnp.int32),
            pltpu.VMEM((NCHUNK, CHUNK), jnp.int32),
            pltpu.VMEM((NCHUNK, CHUNK), jnp.int32),
            pltpu.VMEM((NCHUNK, CHUNK), jnp.int32),
            pltpu.VMEM((CHUNK, WIDE), jnp.float32),
            pltpu.VMEM((CHUNK, WIDE), jnp.float32),
            pltpu.VMEM((CHUNK, WIDE), jnp.float32),
            pltpu.VMEM((NCHUNK, CHUNK), jnp.float32),
            pltpu.SemaphoreType.DMA,
        ],
    )
    def run(u_hbm, i_hbm, j_hbm, ut_hbm, it_hbm, out_hbm,
            u_idx, i_idx, j_idx, uq, iq, jq, ue_v, ie_v, je_v, pred_v, sem):
        wid = lax.axis_index("s") * NUM_CORES + lax.axis_index("c")
        base = wid * B_PER_W
        for c in range(NCHUNK):
            src = pl.ds(base + c * CHUNK, CHUNK)
            pltpu.sync_copy(u_hbm.at[src], u_idx.at[c])
            pltpu.sync_copy(i_hbm.at[src], i_idx.at[c])
            pltpu.sync_copy(j_hbm.at[src], j_idx.at[c])

        def qbody(v, carry):
            s = pl.ds(pl.multiple_of(v * LANES, LANES), LANES)
            for src_ref, dst_ref in ((u_idx, uq), (i_idx, iq), (j_idx, jq)):
                for c in range(NCHUNK):
                    dst_ref[c, s] = lax.shift_right_logical(src_ref[c, s], 2)
            return carry

        lax.fori_loop(0, CHUNK // LANES, qbody, 0)

        lane = lax.iota(jnp.int32, LANES)
        for c in range(NCHUNK):
            cp_u = pltpu.async_copy(ut_hbm.at[uq.at[c]], ue_v, sem)
            cp_i = pltpu.async_copy(it_hbm.at[iq.at[c]], ie_v, sem)
            cp_j = pltpu.async_copy(it_hbm.at[jq.at[c]], je_v, sem)
            cp_u.wait()
            cp_i.wait()
            cp_j.wait()

            def body(g, carry, c=c):
                s = pl.ds(pl.multiple_of(g * LANES, LANES), LANES)
                lrows = pl.multiple_of(g * LANES, LANES) + lane
                cb_u = (u_idx[c, s] & 3) * EDIM
                cb_i = (i_idx[c, s] & 3) * EDIM
                cb_j = (j_idx[c, s] & 3) * EDIM
                acc = jnp.zeros((LANES,), jnp.float32)
                for d in range(EDIM):
                    uev = plsc.load_gather(ue_v, [lrows, cb_u + d])
                    iev = plsc.load_gather(ie_v, [lrows, cb_i + d])
                    jev = plsc.load_gather(je_v, [lrows, cb_j + d])
                    acc = acc + uev * (iev - jev)
                pred_v[c, s] = acc
                return carry

            lax.fori_loop(0, GROUPS, body, 0)
        pltpu.sync_copy(pred_v, out_hbm.at[pl.ds(wid * NCHUNK, NCHUNK)])

    return run(u, i, j, tw_u, tw_i)


def _tc_loss(d2):
    def body(x_ref, o_ref):
        x = x_ref[...]
        ls = jnp.minimum(x, 0.0) - jnp.log(1.0 + jnp.exp(-jnp.abs(x)))
        o_ref[0, 0] = -jnp.sum(ls)

    out = pl.pallas_call(
        body,
        out_shape=jax.ShapeDtypeStruct((1, 1), jnp.float32),
        out_specs=pl.BlockSpec(memory_space=pltpu.SMEM),
    )(d2)
    return out[0, 0]


def kernel(u, i, j, user_embed, item_embed):
    ut_t = user_embed.T
    it_t = item_embed.T
    tw_u, tw_i = _sc_relayout(ut_t, it_t,
                              ut_t[:, TAIL0:], it_t[:, TAIL0:])
    d2 = _sc_pred_diff(u.astype(jnp.int32), i.astype(jnp.int32),
                       j.astype(jnp.int32), tw_u, tw_i)
    return _tc_loss(d2)


# speedup vs baseline: 1.2080x; 1.2080x over previous
import functools

import jax
import jax.numpy as jnp
from jax import lax
from jax.experimental import pallas as pl
from jax.experimental.pallas import tpu as pltpu
from jax.experimental.pallas import tpu_sc as plsc

NUM_CORES = 2
NUM_SUBCORES = 16
LANES = 16
NW = NUM_CORES * NUM_SUBCORES
BATCH = 16384
EDIM = 32
NROW = 1000000
WIDE = 128
PACK = WIDE // EDIM
WROWS = NROW * EDIM // WIDE
CW = 512
NFULL = NROW // CW
TAILW = 128
TAIL0 = NROW - TAILW
B_PER_W = BATCH // NW
CHUNK = 128
NCHUNK = B_PER_W // CHUNK
GROUPS = CHUNK // LANES


def _sc_relayout(ut_t, it_t, tail_u, tail_i):
    mesh = plsc.VectorSubcoreMesh(core_axis_name="c", subcore_axis_name="s")
    out_sds = jax.ShapeDtypeStruct((WROWS, WIDE), jnp.float32)

    @functools.partial(
        pl.kernel,
        out_type=(out_sds, out_sds),
        mesh=mesh,
        compiler_params=pltpu.CompilerParams(
            needs_layout_passes=False, use_tc_tiling_on_sc=True),
        scratch_types=[
            pltpu.VMEM((EDIM, CW), jnp.float32),
            pltpu.VMEM((CW // PACK, WIDE), jnp.float32),
            pltpu.VMEM((EDIM, TAILW), jnp.float32),
            pltpu.SemaphoreType.DMA,
        ],
    )
    def run(ut_hbm, it_hbm, tu_hbm, ti_hbm, uw_hbm, iw_hbm,
            in_v, out_v, in_t, sem):
        wid = lax.axis_index("s") * NUM_CORES + lax.axis_index("c")
        lane = lax.iota(jnp.int32, LANES)
        qbase = lane // PACK
        cmod = (lane % PACK) * EDIM

        for src_hbm, dst_hbm, t_hbm in (
                (ut_hbm, uw_hbm, tu_hbm), (it_hbm, iw_hbm, ti_hbm)):

            def chunk_body(k, carry, src_hbm=src_hbm, dst_hbm=dst_hbm):
                c = k * NW + wid

                @pl.when(c < NFULL)
                def _():
                    r0 = pl.multiple_of(c * CW, CW)
                    pltpu.async_copy(
                        src_hbm.at[:, pl.ds(r0, CW)], in_v, sem).wait()

                    @plsc.parallel_loop(0, CW // LANES, unroll=2)
                    def tbody(g):
                        rowv = 4 * g + qbase
                        s = pl.ds(pl.multiple_of(g * LANES, LANES), LANES)
                        for di in range(EDIM):
                            plsc.store_scatter(out_v, [rowv, cmod + di],
                                               in_v[di, s])
                    pltpu.async_copy(
                        out_v, dst_hbm.at[pl.ds(c * (CW // PACK), CW // PACK)],
                        sem).wait()

                return carry

            lax.fori_loop(0, NFULL // NW + 1, chunk_body, 0)

            @pl.when(wid == 0)
            def _(dst_hbm=dst_hbm, t_hbm=t_hbm):
                pltpu.async_copy(t_hbm, in_t, sem).wait()

                @plsc.parallel_loop(0, TAILW // LANES, unroll=2)
                def tbody(g):
                    rowv = 4 * g + qbase
                    s = pl.ds(pl.multiple_of(g * LANES, LANES), LANES)
                    for di in range(EDIM):
                        plsc.store_scatter(out_v, [rowv, cmod + di],
                                           in_t[di, s])
                pltpu.async_copy(
                    out_v.at[pl.ds(0, TAILW // PACK)],
                    dst_hbm.at[pl.ds(TAIL0 // PACK, TAILW // PACK)], sem).wait()

    return run(ut_t, it_t, tail_u, tail_i)


def _sc_pred_diff(u, i, j, tw_u, tw_i):
    mesh = plsc.VectorSubcoreMesh(core_axis_name="c", subcore_axis_name="s")

    @functools.partial(
        pl.kernel,
        out_type=jax.ShapeDtypeStruct((BATCH // CHUNK, CHUNK), jnp.float32),
        mesh=mesh,
        compiler_params=pltpu.CompilerParams(
            needs_layout_passes=False, use_tc_tiling_on_sc=False),
        scratch_types=[
            pltpu.VMEM((NCHUNK, CHUNK), jnp.int32),
            pltpu.VMEM((NCHUNK, CHUNK), jnp.int32),
            pltpu.VMEM((NCHUNK, CHUNK), jnp.int32),
            pltpu.VMEM((NCHUNK, CHUNK), jnp.int32),
            pltpu.VMEM((NCHUNK, CHUNK), jnp.int32),
            pltpu.VMEM((NCHUNK, CHUNK), jnp.int32),
            pltpu.VMEM((CHUNK, WIDE), jnp.float32),
            pltpu.VMEM((CHUNK, WIDE), jnp.float32),
            pltpu.VMEM((CHUNK, WIDE), jnp.float32),
            pltpu.VMEM((NCHUNK, CHUNK), jnp.float32),
            pltpu.SemaphoreType.DMA,
        ],
    )
    def run(u_hbm, i_hbm, j_hbm, ut_hbm, it_hbm, out_hbm,
            u_idx, i_idx, j_idx, uq, iq, jq, ue_v, ie_v, je_v, pred_v, sem):
        wid = lax.axis_index("s") * NUM_CORES + lax.axis_index("c")
        base = wid * B_PER_W
        for c in range(NCHUNK):
            src = pl.ds(base + c * CHUNK, CHUNK)
            pltpu.sync_copy(u_hbm.at[src], u_idx.at[c])
            pltpu.sync_copy(i_hbm.at[src], i_idx.at[c])
            pltpu.sync_copy(j_hbm.at[src], j_idx.at[c])

        def qbody(v, carry):
            s = pl.ds(pl.multiple_of(v * LANES, LANES), LANES)
            for src_ref, dst_ref in ((u_idx, uq), (i_idx, iq), (j_idx, jq)):
                for c in range(NCHUNK):
                    dst_ref[c, s] = lax.shift_right_logical(src_ref[c, s], 2)
            return carry

        lax.fori_loop(0, CHUNK // LANES, qbody, 0)

        lane = lax.iota(jnp.int32, LANES)
        for c in range(NCHUNK):
            cp_u = pltpu.async_copy(ut_hbm.at[uq.at[c]], ue_v, sem)
            cp_i = pltpu.async_copy(it_hbm.at[iq.at[c]], ie_v, sem)
            cp_j = pltpu.async_copy(it_hbm.at[jq.at[c]], je_v, sem)
            cp_u.wait()
            cp_i.wait()
            cp_j.wait()

            def body(g, carry, c=c):
                s = pl.ds(pl.multiple_of(g * LANES, LANES), LANES)
                lrows = pl.multiple_of(g * LANES, LANES) + lane
                cb_u = (u_idx[c, s] & 3) * EDIM
                cb_i = (i_idx[c, s] & 3) * EDIM
                cb_j = (j_idx[c, s] & 3) * EDIM
                acc = jnp.zeros((LANES,), jnp.float32)
                for d in range(EDIM):
                    uev = plsc.load_gather(ue_v, [lrows, cb_u + d])
                    iev = plsc.load_gather(ie_v, [lrows, cb_i + d])
                    jev = plsc.load_gather(je_v, [lrows, cb_j + d])
                    acc = acc + uev * (iev - jev)
                pred_v[c, s] = acc
                return carry

            lax.fori_loop(0, GROUPS, body, 0)
        pltpu.sync_copy(pred_v, out_hbm.at[pl.ds(wid * NCHUNK, NCHUNK)])

    return run(u, i, j, tw_u, tw_i)


def _tc_loss(d2):
    def body(x_ref, o_ref):
        x = x_ref[...]
        ls = jnp.minimum(x, 0.0) - jnp.log(1.0 + jnp.exp(-jnp.abs(x)))
        o_ref[0, 0] = -jnp.sum(ls)

    out = pl.pallas_call(
        body,
        out_shape=jax.ShapeDtypeStruct((1, 1), jnp.float32),
        out_specs=pl.BlockSpec(memory_space=pltpu.SMEM),
    )(d2)
    return out[0, 0]


def kernel(u, i, j, user_embed, item_embed):
    ut_t = user_embed.T
    it_t = item_embed.T
    tw_u, tw_i = _sc_relayout(ut_t, it_t,
                              ut_t[:, TAIL0:], it_t[:, TAIL0:])
    d2 = _sc_pred_diff(u.astype(jnp.int32), i.astype(jnp.int32),
                       j.astype(jnp.int32), tw_u, tw_i)
    return _tc_loss(d2)


# speedup vs baseline: 1.5372x; 1.2725x over previous
import functools

import jax
import jax.numpy as jnp
from jax import lax
from jax.experimental import pallas as pl
from jax.experimental.pallas import tpu as pltpu
from jax.experimental.pallas import tpu_sc as plsc

NUM_CORES = 2
NUM_SUBCORES = 16
LANES = 16
NW = NUM_CORES * NUM_SUBCORES
BATCH = 16384
EDIM = 32
NROW = 1000000
WIDE = 128
PACK = WIDE // EDIM
WROWS = NROW * EDIM // WIDE
CW = 512
NFULL = NROW // CW
TAILW = 128
TAIL0 = NROW - TAILW
B_PER_W = BATCH // NW
CHUNK = 128
NCHUNK = B_PER_W // CHUNK
GROUPS = CHUNK // LANES


def _sc_relayout(ut_t, it_t, tail_u, tail_i):
    mesh = plsc.VectorSubcoreMesh(core_axis_name="c", subcore_axis_name="s")
    out_sds = jax.ShapeDtypeStruct((WROWS, WIDE), jnp.float32)

    @functools.partial(
        pl.kernel,
        out_type=(out_sds, out_sds),
        mesh=mesh,
        compiler_params=pltpu.CompilerParams(
            needs_layout_passes=False, use_tc_tiling_on_sc=True),
        scratch_types=[
            pltpu.VMEM((EDIM, CW), jnp.float32),
            pltpu.VMEM((EDIM, CW), jnp.float32),
            pltpu.VMEM((CW // PACK, WIDE), jnp.float32),
            pltpu.VMEM((CW // PACK, WIDE), jnp.float32),
            pltpu.VMEM((EDIM, TAILW), jnp.float32),
            pltpu.SemaphoreType.DMA,
            pltpu.SemaphoreType.DMA,
            pltpu.SemaphoreType.DMA,
            pltpu.SemaphoreType.DMA,
        ],
    )
    def run(ut_hbm, it_hbm, tu_hbm, ti_hbm, uw_hbm, iw_hbm,
            in_v0, in_v1, out_v0, out_v1, in_t, isem0, isem1, osem0, osem1):
        wid = lax.axis_index("s") * NUM_CORES + lax.axis_index("c")
        lane = lax.iota(jnp.int32, LANES)
        qbase = lane // PACK
        cmod = (lane % PACK) * EDIM
        in_bufs = (in_v0, in_v1)
        out_bufs = (out_v0, out_v1)
        isems = (isem0, isem1)
        osems = (osem0, osem1)
        NK = NFULL // NW + 1
        assert NK % 2 == 0

        for src_hbm, dst_hbm, t_hbm in (
                (ut_hbm, uw_hbm, tu_hbm), (it_hbm, iw_hbm, ti_hbm)):

            def issue_in(k, buf, sem, src_hbm=src_hbm):
                c = k * NW + wid

                @pl.when(c < NFULL)
                def _():
                    r0 = pl.multiple_of(c * CW, CW)
                    pltpu.async_copy(src_hbm.at[:, pl.ds(r0, CW)], buf, sem)

            def wait_in(k, buf, sem, src_hbm=src_hbm):
                c = k * NW + wid

                @pl.when(c < NFULL)
                def _():
                    pltpu.make_async_copy(
                        src_hbm.at[:, pl.ds(0, CW)], buf, sem).wait()

            def issue_out(k, buf, sem, dst_hbm=dst_hbm):
                c = k * NW + wid

                @pl.when(c < NFULL)
                def _():
                    pltpu.async_copy(
                        buf, dst_hbm.at[pl.ds(c * (CW // PACK), CW // PACK)],
                        sem)

            def wait_out(k, buf, sem, dst_hbm=dst_hbm):
                c = k * NW + wid

                @pl.when((c >= 0) & (c < NFULL))
                def _():
                    pltpu.make_async_copy(
                        buf, dst_hbm.at[pl.ds(0, CW // PACK)], sem).wait()

            def transpose(in_buf, out_buf):
                @plsc.parallel_loop(0, CW // LANES, unroll=2)
                def tbody(g):
                    rowv = 4 * g + qbase
                    s = pl.ds(pl.multiple_of(g * LANES, LANES), LANES)
                    for di in range(EDIM):
                        plsc.store_scatter(out_buf, [rowv, cmod + di],
                                           in_buf[di, s])

            issue_in(0, in_bufs[0], isems[0])
            issue_in(1, in_bufs[1], isems[1])

            def chunk_body(t, carry):
                for p in range(2):
                    k = 2 * t + p
                    c = k * NW + wid

                    @pl.when(c < NFULL)
                    def _(k=k, p=p):
                        wait_in(k, in_bufs[p], isems[p])
                        wait_out(k - 2, out_bufs[p], osems[p])
                        transpose(in_bufs[p], out_bufs[p])
                        issue_out(k, out_bufs[p], osems[p])
                        issue_in(k + 2, in_bufs[p], isems[p])

                return carry

            lax.fori_loop(0, NK // 2, chunk_body, 0)
            wait_out(NK - 2, out_bufs[0], osems[0])
            wait_out(NK - 1, out_bufs[1], osems[1])

            @pl.when(wid == 0)
            def _(dst_hbm=dst_hbm, t_hbm=t_hbm):
                pltpu.async_copy(t_hbm, in_t, isem0).wait()

                @plsc.parallel_loop(0, TAILW // LANES, unroll=2)
                def tbody(g):
                    rowv = 4 * g + qbase
                    s = pl.ds(pl.multiple_of(g * LANES, LANES), LANES)
                    for di in range(EDIM):
                        plsc.store_scatter(out_v0, [rowv, cmod + di],
                                           in_t[di, s])
                pltpu.async_copy(
                    out_v0.at[pl.ds(0, TAILW // PACK)],
                    dst_hbm.at[pl.ds(TAIL0 // PACK, TAILW // PACK)],
                    osem0).wait()

    return run(ut_t, it_t, tail_u, tail_i)


def _sc_pred_diff(u, i, j, tw_u, tw_i):
    mesh = plsc.VectorSubcoreMesh(core_axis_name="c", subcore_axis_name="s")

    @functools.partial(
        pl.kernel,
        out_type=jax.ShapeDtypeStruct((BATCH // CHUNK, CHUNK), jnp.float32),
        mesh=mesh,
        compiler_params=pltpu.CompilerParams(
            needs_layout_passes=False, use_tc_tiling_on_sc=False),
        scratch_types=[
            pltpu.VMEM((NCHUNK, CHUNK), jnp.int32),
            pltpu.VMEM((NCHUNK, CHUNK), jnp.int32),
            pltpu.VMEM((NCHUNK, CHUNK), jnp.int32),
            pltpu.VMEM((NCHUNK, CHUNK), jnp.int32),
            pltpu.VMEM((NCHUNK, CHUNK), jnp.int32),
            pltpu.VMEM((NCHUNK, CHUNK), jnp.int32),
            pltpu.VMEM((CHUNK, WIDE), jnp.float32),
            pltpu.VMEM((CHUNK, WIDE), jnp.float32),
            pltpu.VMEM((CHUNK, WIDE), jnp.float32),
            pltpu.VMEM((NCHUNK, CHUNK), jnp.float32),
            pltpu.SemaphoreType.DMA,
        ],
    )
    def run(u_hbm, i_hbm, j_hbm, ut_hbm, it_hbm, out_hbm,
            u_idx, i_idx, j_idx, uq, iq, jq, ue_v, ie_v, je_v, pred_v, sem):
        wid = lax.axis_index("s") * NUM_CORES + lax.axis_index("c")
        base = wid * B_PER_W
        for c in range(NCHUNK):
            src = pl.ds(base + c * CHUNK, CHUNK)
            pltpu.sync_copy(u_hbm.at[src], u_idx.at[c])
            pltpu.sync_copy(i_hbm.at[src], i_idx.at[c])
            pltpu.sync_copy(j_hbm.at[src], j_idx.at[c])

        def qbody(v, carry):
            s = pl.ds(pl.multiple_of(v * LANES, LANES), LANES)
            for src_ref, dst_ref in ((u_idx, uq), (i_idx, iq), (j_idx, jq)):
                for c in range(NCHUNK):
                    dst_ref[c, s] = lax.shift_right_logical(src_ref[c, s], 2)
            return carry

        lax.fori_loop(0, CHUNK // LANES, qbody, 0)

        lane = lax.iota(jnp.int32, LANES)
        for c in range(NCHUNK):
            cp_u = pltpu.async_copy(ut_hbm.at[uq.at[c]], ue_v, sem)
            cp_i = pltpu.async_copy(it_hbm.at[iq.at[c]], ie_v, sem)
            cp_j = pltpu.async_copy(it_hbm.at[jq.at[c]], je_v, sem)
            cp_u.wait()
            cp_i.wait()
            cp_j.wait()

            def body(g, carry, c=c):
                s = pl.ds(pl.multiple_of(g * LANES, LANES), LANES)
                lrows = pl.multiple_of(g * LANES, LANES) + lane
                cb_u = (u_idx[c, s] & 3) * EDIM
                cb_i = (i_idx[c, s] & 3) * EDIM
                cb_j = (j_idx[c, s] & 3) * EDIM
                acc = jnp.zeros((LANES,), jnp.float32)
                for d in range(EDIM):
                    uev = plsc.load_gather(ue_v, [lrows, cb_u + d])
                    iev = plsc.load_gather(ie_v, [lrows, cb_i + d])
                    jev = plsc.load_gather(je_v, [lrows, cb_j + d])
                    acc = acc + uev * (iev - jev)
                pred_v[c, s] = acc
                return carry

            lax.fori_loop(0, GROUPS, body, 0)
        pltpu.sync_copy(pred_v, out_hbm.at[pl.ds(wid * NCHUNK, NCHUNK)])

    return run(u, i, j, tw_u, tw_i)


def _tc_loss(d2):
    def body(x_ref, o_ref):
        x = x_ref[...]
        ls = jnp.minimum(x, 0.0) - jnp.log(1.0 + jnp.exp(-jnp.abs(x)))
        o_ref[0, 0] = -jnp.sum(ls)

    out = pl.pallas_call(
        body,
        out_shape=jax.ShapeDtypeStruct((1, 1), jnp.float32),
        out_specs=pl.BlockSpec(memory_space=pltpu.SMEM),
    )(d2)
    return out[0, 0]


def kernel(u, i, j, user_embed, item_embed):
    ut_t = user_embed.T
    it_t = item_embed.T
    tw_u, tw_i = _sc_relayout(ut_t, it_t,
                              ut_t[:, TAIL0:], it_t[:, TAIL0:])
    d2 = _sc_pred_diff(u.astype(jnp.int32), i.astype(jnp.int32),
                       j.astype(jnp.int32), tw_u, tw_i)
    return _tc_loss(d2)
